# baseline (speedup 1.0000x reference)
import functools

import jax
import jax.numpy as jnp
from jax import lax
from jax.experimental import pallas as pl
from jax.experimental.pallas import tpu as pltpu
from jax.experimental.pallas import tpu_sc as plsc

_P = 128
_D = 4096
_BLK = 64
_NWORK = 16


def _sc_gather_prompt(prompt_table, prompt_tokens):
    info = plsc.get_sparse_core_info()
    num_cores = info.num_cores
    rows_per_w = _P // _NWORK
    mesh = plsc.VectorSubcoreMesh(core_axis_name="c", subcore_axis_name="s")

    @functools.partial(
        pl.kernel,
        mesh=mesh,
        out_type=jax.ShapeDtypeStruct((_P, _D), jnp.float32),
        scratch_types=[
            pltpu.VMEM((rows_per_w,), jnp.int32),
            pltpu.VMEM((rows_per_w, _D), jnp.float32),
            pltpu.SemaphoreType.DMA,
        ],
    )
    def gather_kernel(table_hbm, tok_hbm, out_hbm, idx_v, rows_v, sem):
        wid = lax.axis_index("s") * num_cores + lax.axis_index("c")

        @pl.when(wid < _NWORK)
        def _():
            base = wid * rows_per_w
            pltpu.sync_copy(tok_hbm.at[pl.ds(base, rows_per_w)], idx_v)
            pltpu.async_copy(table_hbm.at[idx_v], rows_v, sem).wait()
            pltpu.sync_copy(rows_v, out_hbm.at[pl.ds(base, rows_per_w)])

    return gather_kernel(prompt_table, prompt_tokens)


def _tc_copy_body(embedded_input):
    batch, seq, d = embedded_input.shape
    nblk = seq // _BLK

    def body(in_ref, out_ref):
        out_ref[...] = in_ref[...]

    return pl.pallas_call(
        body,
        grid=(nblk,),
        in_specs=[
            pl.BlockSpec((batch, _BLK, d), lambda j: (0, j, 0)),
        ],
        out_specs=pl.BlockSpec((batch, _BLK, d),
                               lambda j: (0, j + _P // _BLK, 0)),
        out_shape=jax.ShapeDtypeStruct((batch, _P + seq, d), jnp.float32),
        compiler_params=pltpu.CompilerParams(
            dimension_semantics=("arbitrary",),
        ),
    )(embedded_input)


def _tc_write_prefix(prompt, buf):
    batch, total, d = buf.shape

    def body(prompt_ref, _buf_ref, out_ref):
        for b in range(batch):
            out_ref[b] = prompt_ref[...]

    return pl.pallas_call(
        body,
        grid=(1,),
        in_specs=[
            pl.BlockSpec((_P, d), lambda j: (0, 0)),
            pl.BlockSpec(memory_space=pltpu.MemorySpace.HBM),
        ],
        out_specs=pl.BlockSpec((batch, _P, d), lambda j: (0, 0, 0)),
        out_shape=jax.ShapeDtypeStruct((batch, total, d), jnp.float32),
        input_output_aliases={1: 0},
    )(prompt, buf)


def kernel(embedded_input, prompt_table, prompt_tokens):
    prompt = _sc_gather_prompt(prompt_table, prompt_tokens)
    buf = _tc_copy_body(embedded_input)
    output = _tc_write_prefix(prompt, buf)
    return (output, _P)


# speedup vs baseline: 1.0037x; 1.0037x over previous
import functools

import jax
import jax.numpy as jnp
from jax import lax
from jax.experimental import pallas as pl
from jax.experimental.pallas import tpu as pltpu
from jax.experimental.pallas import tpu_sc as plsc

_P = 128
_D = 4096
_BLK = 64
_NWORK = 16


def _sc_gather_prompt(prompt_table, prompt_tokens):
    info = plsc.get_sparse_core_info()
    num_cores = info.num_cores
    rows_per_w = _P // _NWORK
    mesh = plsc.VectorSubcoreMesh(core_axis_name="c", subcore_axis_name="s")

    @functools.partial(
        pl.kernel,
        mesh=mesh,
        out_type=jax.ShapeDtypeStruct((_P, _D), jnp.float32),
        scratch_types=[
            pltpu.VMEM((rows_per_w,), jnp.int32),
            pltpu.VMEM((rows_per_w, _D), jnp.float32),
            pltpu.SemaphoreType.DMA,
        ],
    )
    def gather_kernel(table_hbm, tok_hbm, out_hbm, idx_v, rows_v, sem):
        wid = lax.axis_index("s") * num_cores + lax.axis_index("c")

        @pl.when(wid < _NWORK)
        def _():
            base = wid * rows_per_w
            pltpu.sync_copy(tok_hbm.at[pl.ds(base, rows_per_w)], idx_v)
            pltpu.async_copy(table_hbm.at[idx_v], rows_v, sem).wait()
            pltpu.sync_copy(rows_v, out_hbm.at[pl.ds(base, rows_per_w)])

    return gather_kernel(prompt_table, prompt_tokens)


def _tc_copy_body(embedded_input):
    batch, seq, d = embedded_input.shape
    nblk = seq // _BLK

    def body(in_ref, out_ref):
        out_ref[...] = in_ref[...]

    return pl.pallas_call(
        body,
        grid=(nblk,),
        in_specs=[
            pl.BlockSpec((batch, _BLK, d), lambda j: (0, j, 0)),
        ],
        out_specs=pl.BlockSpec((batch, _BLK, d),
                               lambda j: (0, j + _P // _BLK, 0)),
        out_shape=jax.ShapeDtypeStruct((batch, _P + seq, d), jnp.float32),
        compiler_params=pltpu.CompilerParams(
            dimension_semantics=("arbitrary",),
        ),
    )(embedded_input)


def _tc_write_prefix(prompt, buf):
    batch, total, d = buf.shape

    def body(prompt_ref, _buf_ref, out_ref):
        for b in range(batch):
            out_ref[b] = prompt_ref[...]

    return pl.pallas_call(
        body,
        grid=(1,),
        in_specs=[
            pl.BlockSpec((_P, d), lambda j: (0, 0)),
            pl.BlockSpec(memory_space=pltpu.MemorySpace.HBM),
        ],
        out_specs=pl.BlockSpec((batch, _P, d), lambda j: (0, 0, 0)),
        out_shape=jax.ShapeDtypeStruct((batch, total, d), jnp.float32),
        input_output_aliases={1: 0},
    )(prompt, buf)


def kernel(embedded_input, prompt_table, prompt_tokens):
    buf = _tc_copy_body(embedded_input)
    prompt = _sc_gather_prompt(prompt_table, prompt_tokens)
    output = _tc_write_prefix(prompt, buf)
    return (output, _P)


# speedup vs baseline: 1.0238x; 1.0200x over previous
import functools

import jax
import jax.numpy as jnp
from jax import lax
from jax.experimental import pallas as pl
from jax.experimental.pallas import tpu as pltpu
from jax.experimental.pallas import tpu_sc as plsc

_P = 128
_D = 4096
_BLK = 64
_NWORK = 16


def _sc_gather_prompt(prompt_table, prompt_tokens):
    info = plsc.get_sparse_core_info()
    num_cores = info.num_cores
    rows_per_w = _P // _NWORK
    mesh = plsc.VectorSubcoreMesh(core_axis_name="c", subcore_axis_name="s",
                                  num_cores=1)

    @functools.partial(
        pl.kernel,
        mesh=mesh,
        out_type=jax.ShapeDtypeStruct((_P, _D), jnp.float32),
        scratch_types=[
            pltpu.VMEM((rows_per_w,), jnp.int32),
            pltpu.VMEM((rows_per_w, _D), jnp.float32),
            pltpu.SemaphoreType.DMA,
        ],
    )
    def gather_kernel(table_hbm, tok_hbm, out_hbm, idx_v, rows_v, sem):
        wid = lax.axis_index("s") * num_cores + lax.axis_index("c")

        @pl.when(wid < _NWORK)
        def _():
            base = wid * rows_per_w
            pltpu.sync_copy(tok_hbm.at[pl.ds(base, rows_per_w)], idx_v)
            pltpu.async_copy(table_hbm.at[idx_v], rows_v, sem).wait()
            pltpu.sync_copy(rows_v, out_hbm.at[pl.ds(base, rows_per_w)])

    return gather_kernel(prompt_table, prompt_tokens)


def _tc_copy_body(embedded_input):
    batch, seq, d = embedded_input.shape
    nblk = seq // _BLK

    def body(in_ref, out_ref):
        out_ref[...] = in_ref[...]

    return pl.pallas_call(
        body,
        grid=(nblk,),
        in_specs=[
            pl.BlockSpec((batch, _BLK, d), lambda j: (0, j, 0)),
        ],
        out_specs=pl.BlockSpec((batch, _BLK, d),
                               lambda j: (0, j + _P // _BLK, 0)),
        out_shape=jax.ShapeDtypeStruct((batch, _P + seq, d), jnp.float32),
        compiler_params=pltpu.CompilerParams(
            dimension_semantics=("arbitrary",),
        ),
    )(embedded_input)


def _tc_write_prefix(prompt, buf):
    batch, total, d = buf.shape

    def body(prompt_ref, _buf_ref, out_ref):
        for b in range(batch):
            out_ref[b] = prompt_ref[...]

    return pl.pallas_call(
        body,
        grid=(1,),
        in_specs=[
            pl.BlockSpec((_P, d), lambda j: (0, 0)),
            pl.BlockSpec(memory_space=pltpu.MemorySpace.HBM),
        ],
        out_specs=pl.BlockSpec((batch, _P, d), lambda j: (0, 0, 0)),
        out_shape=jax.ShapeDtypeStruct((batch, total, d), jnp.float32),
        input_output_aliases={1: 0},
    )(prompt, buf)


def kernel(embedded_input, prompt_table, prompt_tokens):
    buf = _tc_copy_body(embedded_input)
    prompt = _sc_gather_prompt(prompt_table, prompt_tokens)
    output = _tc_write_prefix(prompt, buf)
    return (output, _P)
